# baseline (speedup 1.0000x reference)
import functools

import jax
import jax.numpy as jnp
from jax import lax
from jax.experimental import pallas as pl
from jax.experimental.pallas import tpu as pltpu
from jax.experimental.pallas import tpu_sc as plsc

N = 10000
NC = 2
NS = 16
CH = 128
NCHUNK = 2560
SPLIT64 = (48, 112)
SPLIT32 = (60, 100)
NBUF = 4
R = 10240
RPT = R // NS
DW = 16


def _zero_rows(ref, nrows, width):
    z = jnp.zeros((16,), jnp.float32)

    def body(i, _):
        for t in range(width // 16):
            ref[i, pl.ds(t * 16, 16)] = z
        return 0

    lax.fori_loop(0, nrows, body, 0)


def _make_edge_pass(D, with_deg, split):
    NCH0, NCH1 = split
    mesh = plsc.VectorSubcoreMesh(core_axis_name="c", subcore_axis_name="s")
    out_type = [jax.ShapeDtypeStruct((NC, R, D), jnp.float32)]
    if with_deg:
        out_type.append(jax.ShapeDtypeStruct((NC, R, DW), jnp.float32))
    nmax = max(NCH0, NCH1)
    scratch = [
        pltpu.VMEM((nmax, CH), jnp.int32),
        pltpu.VMEM((nmax, CH), jnp.int32),
        [pltpu.VMEM((CH, D), jnp.float32) for _ in range(NBUF)],
        pltpu.VMEM_SHARED((R, D), jnp.float32),
        [pltpu.SemaphoreType.DMA for _ in range(NBUF)],
        [pltpu.SemaphoreType.DMA for _ in range(NBUF)],
    ]
    if with_deg:
        scratch += [
            pltpu.VMEM((CH, DW), jnp.float32),
            pltpu.VMEM((CH, DW), jnp.float32),
            pltpu.VMEM_SHARED((R, DW), jnp.float32),
        ]

    def body(y, src0, dst0, src1, dst1, *refs):
        if with_deg:
            (out, dout, src_idx, dst_idx, rows, acc, gsem, ssem,
             ones, zb, dacc) = refs
        else:
            (out, src_idx, dst_idx, rows, acc, gsem, ssem) = refs
        rows_a = rows[0]
        c = lax.axis_index("c")
        s = lax.axis_index("s")

        @pl.when(c == 0)
        def _():
            pltpu.sync_copy(src0.at[s], src_idx.at[pl.ds(0, NCH0)])
            pltpu.sync_copy(dst0.at[s], dst_idx.at[pl.ds(0, NCH0)])

        @pl.when(c == 1)
        def _():
            pltpu.sync_copy(src1.at[s], src_idx.at[pl.ds(0, NCH1)])
            pltpu.sync_copy(dst1.at[s], dst_idx.at[pl.ds(0, NCH1)])

        _zero_rows(rows_a, CH, D)
        for k in range(RPT // CH):
            pltpu.sync_copy(rows_a, acc.at[pl.ds(s * RPT + k * CH, CH)])
        if with_deg:
            _zero_rows(zb, CH, DW)
            one = jnp.full((16,), 1.0, jnp.float32)

            def fill_ones(i, _):
                ones[i, pl.ds(0, 16)] = one
                return 0

            lax.fori_loop(0, CH, fill_ones, 0)
            for k in range(RPT // CH):
                pltpu.sync_copy(zb, dacc.at[pl.ds(s * RPT + k * CH, CH)])
        plsc.subcore_barrier()

        def wait_scatter(k, j):
            pltpu.make_async_copy(rows[k], acc.at[dst_idx.at[j]],
                                  ssem[k]).wait()
            if with_deg:
                pltpu.make_async_copy(ones, dacc.at[dst_idx.at[j]],
                                      ssem[k]).wait()

        def run_edges(nch):
            pltpu.async_copy(y.at[src_idx.at[0]], rows[0], gsem[0])
            pltpu.async_copy(y.at[src_idx.at[1]], rows[1], gsem[1])

            def step(g, _):
                for k in range(NBUF):
                    j = NBUF * g + k
                    pltpu.make_async_copy(y.at[src_idx.at[j]], rows[k],
                                          gsem[k]).wait()
                    pltpu.async_copy(rows[k], acc.at[dst_idx.at[j]], ssem[k],
                                     add=True)
                    if with_deg:
                        pltpu.async_copy(ones, dacc.at[dst_idx.at[j]],
                                         ssem[k], add=True)
                    kn = (k + 2) % NBUF

                    @pl.when(j + 2 < nch)
                    def _():
                        @pl.when(j >= 2)
                        def _():
                            wait_scatter(kn, j - 2)

                        pltpu.async_copy(y.at[src_idx.at[j + 2]], rows[kn],
                                         gsem[kn])

                return 0

            lax.fori_loop(0, nch // NBUF, step, 0)
            for m in range(nch - NBUF, nch):
                wait_scatter(m % NBUF, m)

        @pl.when(c == 0)
        def _():
            run_edges(NCH0)

        @pl.when(c == 1)
        def _():
            run_edges(NCH1)

        plsc.subcore_barrier()

        pltpu.sync_copy(acc.at[pl.ds(s * RPT, RPT)],
                        out.at[c, pl.ds(s * RPT, RPT)])
        if with_deg:
            pltpu.sync_copy(dacc.at[pl.ds(s * RPT, RPT)],
                            dout.at[c, pl.ds(s * RPT, RPT)])

    return pl.kernel(
        body, out_type=out_type, mesh=mesh, scratch_types=scratch,
        compiler_params=pltpu.CompilerParams(use_tc_tiling_on_sc=False))


_edge64_deg = _make_edge_pass(64, True, SPLIT64)
_edge64 = _make_edge_pass(64, False, SPLIT64)
_edge32 = _make_edge_pass(32, False, SPLIT32)


def _tc_in(x, wcat):
    def body(x_ref, w_ref, y_ref, r_ref):
        p = jnp.dot(x_ref[...], w_ref[...], preferred_element_type=jnp.float32)
        y_ref[...] = p[:, :64]
        r_ref[...] = p[:, 64:]

    return pl.pallas_call(
        body,
        out_shape=[jax.ShapeDtypeStruct((N, 64), jnp.float32),
                   jax.ShapeDtypeStruct((N, 64), jnp.float32)],
    )(x, wcat)


def _tc_mid(aA, aB, dA, dB, rprev, b, wcat, d_out):
    def body(aA_r, aB_r, dA_r, dB_r, rp_r, b_r, w_r, y_ref, r_ref):
        invd = 1.0 / jnp.maximum(dA_r[...] + dB_r[...], 1.0)
        h = jnp.maximum((aA_r[...] + aB_r[...]) * invd + rp_r[...] + b_r[...],
                        0.0)
        p = jnp.dot(h, w_r[...], preferred_element_type=jnp.float32)
        y_ref[...] = p[:, :d_out]
        r_ref[...] = p[:, d_out:]

    return pl.pallas_call(
        body,
        out_shape=[jax.ShapeDtypeStruct((N, d_out), jnp.float32),
                   jax.ShapeDtypeStruct((N, wcat.shape[1] - d_out),
                                        jnp.float32)],
    )(aA, aB, dA, dB, rprev, b, wcat)


def _tc_out(aA, aB, dA, dB, rprev, b, wc_pad, bc):
    def body(aA_r, aB_r, dA_r, dB_r, rp_r, b_r, w_r, bc_r, o_ref):
        invd = 1.0 / jnp.maximum(dA_r[...] + dB_r[...], 1.0)
        h = (aA_r[...] + aB_r[...]) * invd + rp_r[...] + b_r[...]
        p = jnp.dot(h, w_r[...], preferred_element_type=jnp.float32)
        o_ref[...] = jax.nn.sigmoid(p + bc_r[...])

    return pl.pallas_call(
        body,
        out_shape=jax.ShapeDtypeStruct((N, 128), jnp.float32),
    )(aA, aB, dA, dB, rprev, b, wc_pad, bc)


def kernel(x, edge_index, Wl1, bl1, Wr1, Wl2, bl2, Wr2, Wl3, bl3, Wr3, Wc, bc):
    ei = edge_index.astype(jnp.int32)
    e = ei.shape[1]
    pad = NCHUNK * CH - e
    src = jnp.concatenate([ei[0], jnp.zeros((pad,), jnp.int32)])
    dst = jnp.concatenate(
        [ei[1], N + (jnp.arange(pad, dtype=jnp.int32) % (R - N))])
    def layout(split):
        n0, n1 = split
        cut = NS * n0 * CH
        return (src[:cut].reshape(NS, n0, CH), dst[:cut].reshape(NS, n0, CH),
                src[cut:].reshape(NS, n1, CH), dst[cut:].reshape(NS, n1, CH))

    e64 = layout(SPLIT64)
    e32 = layout(SPLIT32)

    y1, r1 = _tc_in(x, jnp.concatenate([Wl1, Wr1], axis=1))
    acc1, degs = _edge64_deg(y1, *e64)
    dA = degs[0, :N, 0:1]
    dB = degs[1, :N, 0:1]

    y2, r2 = _tc_mid(acc1[0, :N], acc1[1, :N], dA, dB, r1,
                     bl1.reshape(1, 64), jnp.concatenate([Wl2, Wr2], axis=1),
                     64)
    acc2, = _edge64(y2, *e64)

    y3, r3 = _tc_mid(acc2[0, :N], acc2[1, :N], dA, dB, r2,
                     bl2.reshape(1, 64), jnp.concatenate([Wl3, Wr3], axis=1),
                     32)
    acc3, = _edge32(y3, *e32)

    wc_pad = jnp.pad(Wc, ((0, 0), (0, 127)))
    o = _tc_out(acc3[0, :N], acc3[1, :N], dA, dB, r3,
                bl3.reshape(1, 32), wc_pad, bc.reshape(1, 1))
    return o[:, :1]


# speedup vs baseline: 2.2950x; 2.2950x over previous
import jax
import jax.numpy as jnp
from jax import lax
from jax.experimental import pallas as pl
from jax.experimental.pallas import tpu as pltpu
from jax.experimental.pallas import tpu_sc as plsc

N = 10000
NC = 2
NS = 16
CH = 128
NCHUNK = 2560
NCH = 80
NBUF = 4
NREAL = 2500
LAST_NCH = NREAL - (NC * NS - 1) * NCH
NPT = N // NS
DW = 16

_MESH = plsc.VectorSubcoreMesh(core_axis_name="c", subcore_axis_name="s")
_SC_PARAMS = pltpu.CompilerParams(use_tc_tiling_on_sc=False)


def _zero_rows(ref, nrows, width):
    z = jnp.zeros((16,), jnp.float32)

    def body(i, _):
        for t in range(width // 16):
            ref[i, pl.ds(t * 16, 16)] = z
        return 0

    lax.fori_loop(0, nrows, body, 0)


def _stage_indices(pk_hbm, pk_idx, src_idx, dst_idx, c, s, need_src=True):
    w = s * NC + c
    pltpu.sync_copy(pk_hbm.at[w], pk_idx)

    def unpack(i, _):
        for tt in range(CH // 16):
            v = pk_idx[i, pl.ds(tt * 16, 16)]
            if need_src:
                src_idx[i, pl.ds(tt * 16, 16)] = v & 16383
            dst_idx[i, pl.ds(tt * 16, 16)] = v >> 14
        return 0

    lax.fori_loop(0, NCH, unpack, 0)


def _make_edge_pass(D):
    DH = 32
    nhalf = D // DH
    scratch = [
        pltpu.VMEM((NCH, CH), jnp.int32),
        pltpu.VMEM((NCH, CH), jnp.int32),
        pltpu.VMEM((NCH, CH), jnp.int32),
        [pltpu.VMEM((CH, DH), jnp.float32) for _ in range(NBUF)],
        pltpu.VMEM_SHARED((N, DH), jnp.float32),
        pltpu.VMEM_SHARED((N, DH), jnp.float32),
        [pltpu.SemaphoreType.DMA for _ in range(NBUF)],
        [pltpu.SemaphoreType.DMA for _ in range(NBUF)],
    ]

    def body(y, pk, *refs):
        outs = refs[:nhalf]
        (pk_idx, src_idx, dst_idx, rows, acc, ytab, gsem, ssem) = refs[nhalf:]
        c = lax.axis_index("c")
        s = lax.axis_index("s")
        w = s * NC + c
        nch = jnp.where(w == NC * NS - 1, LAST_NCH, NCH)
        _stage_indices(pk, pk_idx, src_idx, dst_idx, c, s)

        def wait_scatter(k, j):
            pltpu.make_async_copy(rows[k], acc.at[dst_idx.at[j]],
                                  ssem[k]).wait()

        for h in range(nhalf):
            pltpu.sync_copy(y.at[pl.ds(s * NPT, NPT), pl.ds(h * DH, DH)],
                            ytab.at[pl.ds(s * NPT, NPT)])

            _zero_rows(rows[0], CH, DH)
            for k in range(NPT // CH):
                pltpu.sync_copy(rows[0], acc.at[pl.ds(s * NPT + k * CH, CH)])
            pltpu.sync_copy(
                rows[0].at[pl.ds(0, NPT % CH)],
                acc.at[pl.ds(s * NPT + (NPT // CH) * CH, NPT % CH)])
            plsc.subcore_barrier()

            pltpu.async_copy(ytab.at[src_idx.at[0]], rows[0], gsem[0])
            pltpu.async_copy(ytab.at[src_idx.at[1]], rows[1], gsem[1])

            def step(g, _):
                for k in range(NBUF):
                    j = NBUF * g + k
                    pltpu.make_async_copy(ytab.at[src_idx.at[j]], rows[k],
                                          gsem[k]).wait()
                    pltpu.async_copy(rows[k], acc.at[dst_idx.at[j]], ssem[k],
                                     add=True)
                    kn = (k + 2) % NBUF

                    @pl.when(j + 2 < nch)
                    def _():
                        @pl.when(j >= 2)
                        def _():
                            wait_scatter(kn, j - 2)

                        pltpu.async_copy(ytab.at[src_idx.at[j + 2]],
                                         rows[kn], gsem[kn])

                return 0

            lax.fori_loop(0, nch // NBUF, step, 0)
            for i in range(NBUF):
                wait_scatter(i, nch - NBUF + i)
            plsc.subcore_barrier()

            pltpu.sync_copy(acc.at[pl.ds(s * NPT, NPT)],
                            outs[h].at[c, pl.ds(s * NPT, NPT)])

    return pl.kernel(
        body,
        out_type=[jax.ShapeDtypeStruct((NC, N, DH), jnp.float32)
                  for _ in range(nhalf)],
        mesh=_MESH, scratch_types=scratch, compiler_params=_SC_PARAMS)


def _make_deg_pass():
    scratch = [
        pltpu.VMEM((NCH, CH), jnp.int32),
        pltpu.VMEM((NCH, CH), jnp.int32),
        pltpu.VMEM((CH, DW), jnp.float32),
        pltpu.VMEM((CH, DW), jnp.float32),
        pltpu.VMEM_SHARED((N, DW), jnp.float32),
        [pltpu.SemaphoreType.DMA for _ in range(NBUF)],
    ]

    def body(pk, dout, pk_idx, dst_idx, ones, zb, dacc, ssem):
        c = lax.axis_index("c")
        s = lax.axis_index("s")
        w = s * NC + c
        nch = jnp.where(w == NC * NS - 1, LAST_NCH, NCH)
        _stage_indices(pk, pk_idx, dst_idx, dst_idx, c, s, need_src=False)

        _zero_rows(zb, CH, DW)
        one = jnp.full((16,), 1.0, jnp.float32)

        def fill_ones(i, _):
            ones[i, pl.ds(0, 16)] = one
            return 0

        lax.fori_loop(0, CH, fill_ones, 0)
        for k in range(NPT // CH):
            pltpu.sync_copy(zb, dacc.at[pl.ds(s * NPT + k * CH, CH)])
        pltpu.sync_copy(zb.at[pl.ds(0, NPT % CH)],
                        dacc.at[pl.ds(s * NPT + (NPT // CH) * CH, NPT % CH)])
        plsc.subcore_barrier()

        def wait_scatter(k, j):
            pltpu.make_async_copy(ones, dacc.at[dst_idx.at[j]],
                                  ssem[k]).wait()

        def step(g, _):
            for k in range(NBUF):
                j = NBUF * g + k

                @pl.when(j >= NBUF)
                def _():
                    wait_scatter(k, j - NBUF)

                pltpu.async_copy(ones, dacc.at[dst_idx.at[j]], ssem[k],
                                 add=True)
            return 0

        lax.fori_loop(0, nch // NBUF, step, 0)
        for i in range(NBUF):
            wait_scatter(i, nch - NBUF + i)
        plsc.subcore_barrier()

        pltpu.sync_copy(dacc.at[pl.ds(s * NPT, NPT)],
                        dout.at[c, pl.ds(s * NPT, NPT)])

    return pl.kernel(
        body, out_type=[jax.ShapeDtypeStruct((NC, N, DW), jnp.float32)],
        mesh=_MESH, scratch_types=scratch, compiler_params=_SC_PARAMS)


_edge64 = _make_edge_pass(64)
_edge32 = _make_edge_pass(32)
_deg_pass = _make_deg_pass()


def _tc_in(x, wcat):
    def body(x_ref, w_ref, y_ref, r_ref):
        p = jnp.dot(x_ref[...], w_ref[...], preferred_element_type=jnp.float32)
        y_ref[...] = p[:, :64]
        r_ref[...] = p[:, 64:]

    return pl.pallas_call(
        body,
        out_shape=[jax.ShapeDtypeStruct((N, 64), jnp.float32),
                   jax.ShapeDtypeStruct((N, 64), jnp.float32)],
    )(x, wcat)


def _agg(part_refs, deg_r):
    invd = 1.0 / jnp.maximum(deg_r[0, :, 0:1] + deg_r[1, :, 0:1], 1.0)
    a = jnp.concatenate([pr[0] + pr[1] for pr in part_refs], axis=1)
    return a * invd


def _tc_mid(parts, degs, rprev, b, wcat, d_out):
    n_p = len(parts)

    def body(*refs):
        p_refs = refs[:n_p]
        deg_r, rp_r, b_r, w_r, y_ref, r_ref = refs[n_p:]
        h = jnp.maximum(_agg(p_refs, deg_r) + rp_r[...] + b_r[...], 0.0)
        p = jnp.dot(h, w_r[...], preferred_element_type=jnp.float32)
        y_ref[...] = p[:, :d_out]
        r_ref[...] = p[:, d_out:]

    return pl.pallas_call(
        body,
        out_shape=[jax.ShapeDtypeStruct((N, d_out), jnp.float32),
                   jax.ShapeDtypeStruct((N, wcat.shape[1] - d_out),
                                        jnp.float32)],
    )(*parts, degs, rprev, b, wcat)


def _tc_out(parts, degs, rprev, b, wc_pad, bc):
    n_p = len(parts)

    def body(*refs):
        p_refs = refs[:n_p]
        deg_r, rp_r, b_r, w_r, bc_r, o_ref = refs[n_p:]
        h = _agg(p_refs, deg_r) + rp_r[...] + b_r[...]
        p = jnp.dot(h, w_r[...], preferred_element_type=jnp.float32)
        o_ref[...] = jax.nn.sigmoid(p + bc_r[...])

    return pl.pallas_call(
        body,
        out_shape=jax.ShapeDtypeStruct((N, 128), jnp.float32),
    )(*parts, degs, rprev, b, wc_pad, bc)


def kernel(x, edge_index, Wl1, bl1, Wr1, Wl2, bl2, Wr2, Wl3, bl3, Wr3, Wc, bc):
    ei = edge_index.astype(jnp.int32)
    e = ei.shape[1]
    pad = NCHUNK * CH - e
    src = jnp.concatenate([ei[0], jnp.zeros((pad,), jnp.int32)])
    dst = jnp.concatenate([ei[1], jnp.zeros((pad,), jnp.int32)])
    pk = ((dst << 14) | src).reshape(NC * NS, NCH, CH)

    degs, = _deg_pass(pk)

    y1, r1 = _tc_in(x, jnp.concatenate([Wl1, Wr1], axis=1))
    acc1 = _edge64(y1, pk)

    y2, r2 = _tc_mid(acc1, degs, r1, bl1.reshape(1, 64),
                     jnp.concatenate([Wl2, Wr2], axis=1), 64)
    acc2 = _edge64(y2, pk)

    y3, r3 = _tc_mid(acc2, degs, r2, bl2.reshape(1, 64),
                     jnp.concatenate([Wl3, Wr3], axis=1), 32)
    acc3 = _edge32(y3, pk)

    wc_pad = jnp.pad(Wc, ((0, 0), (0, 127)))
    o = _tc_out(acc3, degs, r3, bl3.reshape(1, 32), wc_pad,
                bc.reshape(1, 1))
    return o[:, :1]
